# Initial kernel scaffold; baseline (speedup 1.0000x reference)
import jax
import jax.numpy as jnp
from jax.experimental import pallas as pl


def kernel(seq, embed, ff_w1, ff_b1, ff_w2, ff_b2, ln_g, ln_b, base_w1, base_b1, base_w2, base_b2, out_w, out_b):
    raise NotImplementedError("write your pallas kernel here")


# speedup vs baseline: 19.0234x; 19.0234x over previous
import functools

import jax
import jax.numpy as jnp
from jax.experimental import pallas as pl
from jax.experimental.pallas import tpu as pltpu

H = 64
INNER = 32
VOCAB = 64
EMA, LR, THRESH, EPS = 0.9, 0.05, 1.5, 1e-5
CL = 64


def _rb(t):
    return t.astype(jnp.bfloat16).astype(jnp.float32)


def _fused_kernel(seq_ref, embedT_ref, ffw1_ref, ffb1_ref, ffw2_ref,
                  ffb2_ref, lng_ref, lnb_ref, bw1_ref, bb1_ref, bw2t_ref,
                  bb2_ref, outw_ref, outb_ref, o_ref,
                  hs_scr, w1_s, w2_s, b1_s, b2_s, ema_s, *, nchunk, b, t_valid):
    j = pl.program_id(0)
    ct = CL // 2

    @pl.when(j == 0)
    def _init():
        w1_s[...] = bw1_ref[...]
        w2_s[...] = bw2t_ref[...]
        b1_s[...] = bb1_ref[...]
        b2_s[...] = bb2_ref[...]
        ema_s[...] = jnp.ones_like(ema_s)

    n = CL * b
    seq_row = seq_ref[0, 0]
    iota_v = jax.lax.broadcasted_iota(jnp.int32, (VOCAB, n), 0)
    oh = (iota_v == jnp.broadcast_to(seq_row, (VOCAB, n))).astype(jnp.float32)
    e = jnp.dot(embedT_ref[...], oh, preferred_element_type=jnp.float32,
                precision=jax.lax.Precision.HIGHEST)
    x1 = jnp.maximum(
        jnp.dot(ffw1_ref[...].astype(jnp.bfloat16), e.astype(jnp.bfloat16),
                preferred_element_type=jnp.float32)
        + ffb1_ref[...], 0.0)
    x = e + jnp.dot(ffw2_ref[...].astype(jnp.bfloat16),
                    x1.astype(jnp.bfloat16),
                    preferred_element_type=jnp.float32) + ffb2_ref[...]
    mu = jnp.mean(x, axis=0, keepdims=True)
    xc = x - mu
    var = jnp.mean(xc * xc, axis=0, keepdims=True)
    hs = xc / jnp.sqrt(var + EPS) * lng_ref[...] + lnb_ref[...]

    for l in range(CL):
        hs_scr[l] = hs[:, l * b:(l + 1) * b]

    def step(p, _):
        xk = hs_scr[2 * p]
        v = hs_scr[2 * p + 1]
        w1 = w1_s[...]
        w2 = w2_s[...]
        w1r = _rb(w1)
        w2r = _rb(w2)
        xkr = _rb(xk)
        z = jnp.sum(w1r * xkr[None, :, :], axis=1) + b1_s[...]
        a = jnp.maximum(z, 0.0)
        pred = jnp.sum(w2r * _rb(a)[:, None, :], axis=0) + b2_s[...]
        d = (2.0 / H) * (pred - v)
        dz = jnp.sum(w2r * _rb(d)[None, :, :], axis=1)
        dz = jnp.where(z > 0, dz, 0.0)
        g1 = dz[:, None, :] * xk[None, :, :]
        g2 = a[:, None, :] * d[None, :, :]
        s_w1 = jnp.sum(g1 * g1, axis=(0, 1), keepdims=True)
        s_b1 = jnp.sum(dz * dz, axis=0, keepdims=True)
        s_w2 = jnp.sum(g2 * g2, axis=(0, 1), keepdims=True)
        s_b2 = jnp.sum(d * d, axis=0, keepdims=True)
        gsq = ((s_w1[0] + s_b1) + s_w2[0]) + s_b2
        em = EMA * ema_s[...] + (1.0 - EMA) * gsq
        ema_s[...] = em
        ratio = gsq / jnp.maximum(em, 1e-8)
        t_glob = j * ct + p
        upd = (ratio > THRESH) & (t_glob < t_valid)
        lrm = jnp.where(upd, LR, 0.0)
        w1_s[...] = w1 - lrm[None] * g1
        b1_s[...] = b1_s[...] - lrm * dz
        w2_s[...] = w2 - lrm[None] * g2
        b2_s[...] = b2_s[...] - lrm * d
        return 0

    jax.lax.fori_loop(0, ct, step, 0)

    @pl.when(j == nchunk - 1)
    def _final():
        lh = hs_scr[CL - 1]
        zf = jnp.sum(_rb(w1_s[...]) * _rb(lh)[None, :, :], axis=1) + b1_s[...]
        af = jnp.maximum(zf, 0.0)
        ctx = jnp.sum(_rb(w2_s[...]) * _rb(af)[:, None, :], axis=0) + b2_s[...]
        o_ref[...] = jnp.dot(outw_ref[...].astype(jnp.bfloat16),
                             ctx.astype(jnp.bfloat16),
                             preferred_element_type=jnp.float32) + outb_ref[...]


def _build(nchunk, b, t_valid, interpret=False):
    grid = (nchunk,)
    full = lambda shape: pl.BlockSpec(shape, lambda j: tuple(0 for _ in shape))
    in_specs = [
        pl.BlockSpec((1, 1, 1, CL * b), lambda j: (0, j, 0, 0)),
        full((H, VOCAB)),
        full((2 * H, H)),
        full((2 * H, 1)),
        full((H, 2 * H)),
        full((H, 1)),
        full((H, 1)),
        full((H, 1)),
        full((INNER, H, b)),
        full((INNER, b)),
        full((INNER, H, b)),
        full((H, b)),
        full((VOCAB, H)),
        full((VOCAB, 1)),
    ]
    return pl.pallas_call(
        functools.partial(_fused_kernel, nchunk=nchunk, b=b, t_valid=t_valid),
        grid=grid,
        in_specs=in_specs,
        out_specs=pl.BlockSpec((VOCAB, b), lambda j: (0, 0)),
        out_shape=jax.ShapeDtypeStruct((VOCAB, b), jnp.float32),
        scratch_shapes=[
            pltpu.VMEM((CL, H, b), jnp.float32),
            pltpu.VMEM((INNER, H, b), jnp.float32),
            pltpu.VMEM((INNER, H, b), jnp.float32),
            pltpu.VMEM((INNER, b), jnp.float32),
            pltpu.VMEM((H, b), jnp.float32),
            pltpu.VMEM((1, b), jnp.float32),
        ],
        compiler_params=pltpu.CompilerParams(
            dimension_semantics=("arbitrary",),
            vmem_limit_bytes=100 * 1024 * 1024,
        ),
        name="surprise_ttt_fused",
        interpret=interpret,
    )


def kernel(seq, embed, ff_w1, ff_b1, ff_w2, ff_b2, ln_g, ln_b,
           base_w1, base_b1, base_w2, base_b2, out_w, out_b,
           interpret=False):
    b, l = seq.shape
    assert l % CL == 0
    nchunk = l // CL
    t_valid = l // 2 - 1

    seq_c = jnp.transpose(seq).reshape(nchunk, 1, CL * b)
    seq_c = seq_c.reshape(nchunk, 1, 1, CL * b).transpose(1, 0, 2, 3)
    col = lambda v: v.reshape(-1, 1)
    bw1_b = jnp.broadcast_to(base_w1[:, :, None], (INNER, H, b))
    bw2t_b = jnp.broadcast_to(base_w2.T[:, :, None], (INNER, H, b))
    bb1_b = jnp.broadcast_to(base_b1[:, None], (INNER, b))
    bb2_b = jnp.broadcast_to(base_b2[:, None], (H, b))

    out_vb = _build(nchunk, b, t_valid, interpret=interpret)(
        seq_c, embed.T, ff_w1, col(ff_b1), ff_w2, col(ff_b2),
        col(ln_g), col(ln_b), bw1_b, bb1_b, bw2t_b, bb2_b,
        out_w, col(out_b))
    return out_vb.T


# speedup vs baseline: 22.5561x; 1.1857x over previous
import functools

import jax
import jax.numpy as jnp
from jax.experimental import pallas as pl
from jax.experimental.pallas import tpu as pltpu

H = 64
INNER = 32
VOCAB = 64
EMA, LR, THRESH, EPS = 0.9, 0.05, 1.5, 1e-5
CL = 64


def _rb(t):
    return t.astype(jnp.bfloat16).astype(jnp.float32)


def _fused_kernel(seq_ref, embedT_ref, ffw1_ref, ffb1_ref, ffw2_ref,
                  ffb2_ref, lng_ref, lnb_ref, bw1_ref, bb1_ref, bw2t_ref,
                  bb2_ref, outw_ref, outb_ref, o_ref,
                  hs_scr, w1_s, w2_s, b1_s, b2_s, ema_s, *, nchunk, b, t_valid):
    j = pl.program_id(0)
    ct = CL // 2

    @pl.when(j == 0)
    def _init():
        w1_s[...] = bw1_ref[...]
        w2_s[...] = bw2t_ref[...]
        b1_s[...] = bb1_ref[...]
        b2_s[...] = bb2_ref[...]
        ema_s[...] = jnp.ones_like(ema_s)

    n = CL * b
    seq_row = seq_ref[0, 0]
    iota_v = jax.lax.broadcasted_iota(jnp.int32, (VOCAB, n), 0)
    oh = (iota_v == jnp.broadcast_to(seq_row, (VOCAB, n))).astype(jnp.float32)
    e = jnp.dot(embedT_ref[...], oh, preferred_element_type=jnp.float32,
                precision=jax.lax.Precision.HIGHEST)
    x1 = jnp.maximum(
        jnp.dot(ffw1_ref[...].astype(jnp.bfloat16), e.astype(jnp.bfloat16),
                preferred_element_type=jnp.float32)
        + ffb1_ref[...], 0.0)
    x = e + jnp.dot(ffw2_ref[...].astype(jnp.bfloat16),
                    x1.astype(jnp.bfloat16),
                    preferred_element_type=jnp.float32) + ffb2_ref[...]
    mu = jnp.mean(x, axis=0, keepdims=True)
    xc = x - mu
    var = jnp.mean(xc * xc, axis=0, keepdims=True)
    hs = xc / jnp.sqrt(var + EPS) * lng_ref[...] + lnb_ref[...]

    for l in range(CL):
        hs_scr[l] = hs[:, l * b:(l + 1) * b]

    def step(p, _):
        xk = hs_scr[2 * p]
        v = hs_scr[2 * p + 1]
        w1 = w1_s[...]
        w2 = w2_s[...]
        w1r = _rb(w1)
        w2r = _rb(w2)
        xkr = _rb(xk)
        z = jnp.sum(w1r * xkr[None, :, :], axis=1) + b1_s[...]
        a = jnp.maximum(z, 0.0)
        pred = jnp.sum(w2r * _rb(a)[:, None, :], axis=0) + b2_s[...]
        d = (2.0 / H) * (pred - v)
        dz = jnp.sum(w2r * _rb(d)[None, :, :], axis=1)
        dz = jnp.where(z > 0, dz, 0.0)
        g1 = dz[:, None, :] * xk[None, :, :]
        g2 = a[:, None, :] * d[None, :, :]
        s_w1 = jnp.sum(g1 * g1, axis=(0, 1), keepdims=True)
        s_b1 = jnp.sum(dz * dz, axis=0, keepdims=True)
        s_w2 = jnp.sum(g2 * g2, axis=(0, 1), keepdims=True)
        s_b2 = jnp.sum(d * d, axis=0, keepdims=True)
        gsq = ((s_w1[0] + s_b1) + s_w2[0]) + s_b2
        em = EMA * ema_s[...] + (1.0 - EMA) * gsq
        ema_s[...] = em
        ratio = gsq / jnp.maximum(em, 1e-8)
        t_glob = j * ct + p
        upd = (ratio > THRESH) & (t_glob < t_valid)
        lrm = jnp.where(upd, LR, 0.0)
        w1_s[...] = w1 - lrm[None] * g1
        b1_s[...] = b1_s[...] - lrm * dz
        w2_s[...] = w2 - lrm[None] * g2
        b2_s[...] = b2_s[...] - lrm * d
        return 0

    jax.lax.fori_loop(0, ct, step, 0, unroll=16)

    @pl.when(j == nchunk - 1)
    def _final():
        lh = hs_scr[CL - 1]
        zf = jnp.sum(_rb(w1_s[...]) * _rb(lh)[None, :, :], axis=1) + b1_s[...]
        af = jnp.maximum(zf, 0.0)
        ctx = jnp.sum(_rb(w2_s[...]) * _rb(af)[:, None, :], axis=0) + b2_s[...]
        o_ref[...] = jnp.dot(outw_ref[...].astype(jnp.bfloat16),
                             ctx.astype(jnp.bfloat16),
                             preferred_element_type=jnp.float32) + outb_ref[...]


def _build(nchunk, b, t_valid, interpret=False):
    grid = (nchunk,)
    full = lambda shape: pl.BlockSpec(shape, lambda j: tuple(0 for _ in shape))
    in_specs = [
        pl.BlockSpec((1, 1, 1, CL * b), lambda j: (0, j, 0, 0)),
        full((H, VOCAB)),
        full((2 * H, H)),
        full((2 * H, 1)),
        full((H, 2 * H)),
        full((H, 1)),
        full((H, 1)),
        full((H, 1)),
        full((INNER, H, b)),
        full((INNER, b)),
        full((INNER, H, b)),
        full((H, b)),
        full((VOCAB, H)),
        full((VOCAB, 1)),
    ]
    return pl.pallas_call(
        functools.partial(_fused_kernel, nchunk=nchunk, b=b, t_valid=t_valid),
        grid=grid,
        in_specs=in_specs,
        out_specs=pl.BlockSpec((VOCAB, b), lambda j: (0, 0)),
        out_shape=jax.ShapeDtypeStruct((VOCAB, b), jnp.float32),
        scratch_shapes=[
            pltpu.VMEM((CL, H, b), jnp.float32),
            pltpu.VMEM((INNER, H, b), jnp.float32),
            pltpu.VMEM((INNER, H, b), jnp.float32),
            pltpu.VMEM((INNER, b), jnp.float32),
            pltpu.VMEM((H, b), jnp.float32),
            pltpu.VMEM((1, b), jnp.float32),
        ],
        compiler_params=pltpu.CompilerParams(
            dimension_semantics=("arbitrary",),
            vmem_limit_bytes=100 * 1024 * 1024,
        ),
        name="surprise_ttt_fused",
        interpret=interpret,
    )


def kernel(seq, embed, ff_w1, ff_b1, ff_w2, ff_b2, ln_g, ln_b,
           base_w1, base_b1, base_w2, base_b2, out_w, out_b,
           interpret=False):
    b, l = seq.shape
    assert l % CL == 0
    nchunk = l // CL
    t_valid = l // 2 - 1

    seq_c = jnp.transpose(seq).reshape(nchunk, 1, CL * b)
    seq_c = seq_c.reshape(nchunk, 1, 1, CL * b).transpose(1, 0, 2, 3)
    col = lambda v: v.reshape(-1, 1)
    bw1_b = jnp.broadcast_to(base_w1[:, :, None], (INNER, H, b))
    bw2t_b = jnp.broadcast_to(base_w2.T[:, :, None], (INNER, H, b))
    bb1_b = jnp.broadcast_to(base_b1[:, None], (INNER, b))
    bb2_b = jnp.broadcast_to(base_b2[:, None], (H, b))

    out_vb = _build(nchunk, b, t_valid, interpret=interpret)(
        seq_c, embed.T, ff_w1, col(ff_b1), ff_w2, col(ff_b2),
        col(ln_g), col(ln_b), bw1_b, bb1_b, bw2t_b, bb2_b,
        out_w, col(out_b))
    return out_vb.T
